# baseline (speedup 1.0000x reference)
import functools

import jax
import jax.numpy as jnp
from jax import lax
from jax.experimental import pallas as pl
from jax.experimental.pallas import tpu as pltpu
from jax.experimental.pallas import tpu_sc as plsc

_B, _S, _D = 4, 8192, 1024
_NW = 32
_ROWS_W = _S // _NW
_C = 16
_NCH = _ROWS_W // _C
_LANES = 16
_SLICES = _D // _LANES


def _sc_body(in_hbm, pos_hbm, out_hbm, pos_v, io, sem_in, sem_out):
    wid = lax.axis_index("s") * 2 + lax.axis_index("c")
    base = wid * _ROWS_W

    def chunk_body(ci, _):
        row0 = base + ci * _C

        for b in range(_B):
            pltpu.async_copy(
                in_hbm.at[pl.ds(b * _S + row0, _C)], io.at[b], sem_in[b]
            )
        pltpu.sync_copy(pos_hbm.at[pl.ds(row0, _C)], pos_v)

        for b in range(_B):
            pltpu.make_async_copy(
                in_hbm.at[pl.ds(b * _S + row0, _C)], io.at[b], sem_in[b]
            ).wait()

            def row_body(r, _, b=b):
                for j in range(_SLICES):
                    sl = pl.ds(j * _LANES, _LANES)
                    plsc.addupdate(io.at[b, r, sl], pos_v[r, sl])
                return 0

            lax.fori_loop(0, _C, row_body, 0)

            pltpu.async_copy(
                io.at[b], out_hbm.at[pl.ds(b * _S + row0, _C)], sem_out[b]
            )

        for b in range(_B):
            pltpu.make_async_copy(
                io.at[b], out_hbm.at[pl.ds(b * _S + row0, _C)], sem_out[b]
            ).wait()
        return 0

    lax.fori_loop(0, _NCH, chunk_body, 0)


_sc_add = functools.partial(
    pl.kernel,
    mesh=plsc.VectorSubcoreMesh(core_axis_name="c", subcore_axis_name="s"),
    out_type=jax.ShapeDtypeStruct((_B * _S, _D), jnp.float32),
    scratch_types=[
        pltpu.VMEM((_C, _D), jnp.float32),
        pltpu.VMEM((_B, _C, _D), jnp.float32),
        [pltpu.SemaphoreType.DMA] * _B,
        [pltpu.SemaphoreType.DMA] * _B,
    ],
)(_sc_body)


def kernel(inputs, pos_table):
    B, S, D = inputs.shape
    flat = inputs.reshape(B * S, D)
    out = _sc_add(flat, pos_table)
    return out.reshape(B, S, D)
